# Initial kernel scaffold; baseline (speedup 1.0000x reference)
import jax
import jax.numpy as jnp
from jax.experimental import pallas as pl


def kernel(hidden_states, gate_weight, Wg, Wu, Wd, Wsg, Wsu, Wsd):
    raise NotImplementedError("write your pallas kernel here")


# speedup vs baseline: 5.9452x; 5.9452x over previous
import jax
import jax.numpy as jnp
from jax.experimental import pallas as pl

_NUM_EXPERTS = 64
_D_MODEL = 8
_D_FF = 16
_TILE = 2048


def _moe_tile_kernel(x_ref, gwt_ref, wall_ref, wsgu_ref, wsdt_ref, out_ref):
    x = x_ref[...]

    logits = jnp.dot(x, gwt_ref[...], preferred_element_type=jnp.float32)
    m = jnp.max(logits, axis=-1, keepdims=True)
    w = 1.0 / jnp.sum(jnp.exp(logits - m), axis=-1, keepdims=True)
    eq = logits == m
    lane = jax.lax.broadcasted_iota(jnp.int32, logits.shape, 1)
    am = jnp.min(jnp.where(eq, lane, _NUM_EXPERTS), axis=-1, keepdims=True)
    onehot = (lane == am).astype(jnp.float32)

    wt = jnp.dot(onehot, wall_ref[...], preferred_element_type=jnp.float32)
    wg_t = wt[:, 0:128].reshape(-1, _D_MODEL, _D_FF)
    wu_t = wt[:, 128:256].reshape(-1, _D_MODEL, _D_FF)
    wd_t = wt[:, 256:384].reshape(-1, _D_MODEL, _D_FF)

    xe = x[:, :, None]
    g = jnp.sum(xe * wg_t, axis=1)
    u = jnp.sum(xe * wu_t, axis=1)
    h = (g * jax.nn.sigmoid(g)) * u * w
    routed = jnp.sum(h[:, None, :] * wd_t, axis=2)

    gu = jnp.dot(x, wsgu_ref[...], preferred_element_type=jnp.float32)
    gs = gu[:, :_D_FF]
    us = gu[:, _D_FF:]
    hs = (gs * jax.nn.sigmoid(gs)) * us
    shared = jnp.dot(hs, wsdt_ref[...], preferred_element_type=jnp.float32)

    out_ref[...] = routed + shared


def kernel(hidden_states, gate_weight, Wg, Wu, Wd, Wsg, Wsu, Wsd):
    orig_shape = hidden_states.shape
    x2 = hidden_states.reshape(-1, _D_MODEL)
    t = x2.shape[0]

    wg_p = Wg.transpose(0, 2, 1).reshape(_NUM_EXPERTS, _D_MODEL * _D_FF)
    wu_p = Wu.transpose(0, 2, 1).reshape(_NUM_EXPERTS, _D_MODEL * _D_FF)
    wd_p = Wd.reshape(_NUM_EXPERTS, _D_MODEL * _D_FF)
    w_all = jnp.concatenate([wg_p, wu_p, wd_p], axis=1)
    wsgu = jnp.concatenate([Wsg.T, Wsu.T], axis=1)

    grid = (t // _TILE,)
    out = pl.pallas_call(
        _moe_tile_kernel,
        grid=grid,
        in_specs=[
            pl.BlockSpec((_TILE, _D_MODEL), lambda i: (i, 0)),
            pl.BlockSpec((_D_MODEL, _NUM_EXPERTS), lambda i: (0, 0)),
            pl.BlockSpec((_NUM_EXPERTS, 3 * _D_MODEL * _D_FF), lambda i: (0, 0)),
            pl.BlockSpec((_D_MODEL, 2 * _D_FF), lambda i: (0, 0)),
            pl.BlockSpec((_D_FF, _D_MODEL), lambda i: (0, 0)),
        ],
        out_specs=pl.BlockSpec((_TILE, _D_MODEL), lambda i: (i, 0)),
        out_shape=jax.ShapeDtypeStruct((t, _D_MODEL), jnp.float32),
    )(x2, gate_weight.T, w_all, wsgu, Wsd.T)
    return out.reshape(orig_shape)


# speedup vs baseline: 32.0623x; 5.3929x over previous
import jax
import jax.numpy as jnp
import numpy as np
from jax.experimental import pallas as pl

_NUM_EXPERTS = 64
_D_MODEL = 8
_D_FF = 16
_TILE = 2048


def _moe_tile_kernel(x_ref, gwt_ref, wall_ref, r_ref, s_ref, q_ref, sd_ref,
                     wsgu_ref, wsdt_ref, out_ref):
    f32 = jnp.float32
    x = x_ref[...]

    logits = jnp.dot(x, gwt_ref[...], preferred_element_type=f32)
    m = jnp.max(logits, axis=-1, keepdims=True)
    w = 1.0 / jnp.sum(jnp.exp(logits - m), axis=-1, keepdims=True)
    lane = jax.lax.broadcasted_iota(jnp.int32, logits.shape, 1)
    am = jnp.min(jnp.where(logits == m, lane, _NUM_EXPERTS), axis=-1, keepdims=True)
    onehot = (lane == am).astype(f32)

    wt = jnp.dot(onehot, wall_ref[...], preferred_element_type=f32)
    xx = jnp.dot(x, r_ref[...], preferred_element_type=f32)
    g = jnp.dot(xx * wt[:, 0:128], s_ref[...], preferred_element_type=f32)
    u = jnp.dot(xx * wt[:, 128:256], s_ref[...], preferred_element_type=f32)
    h = (g * jax.nn.sigmoid(g)) * u * w
    hh = jnp.dot(h, q_ref[...], preferred_element_type=f32)
    routed = jnp.dot(hh * wt[:, 256:384], sd_ref[...], preferred_element_type=f32)

    gu = jnp.dot(x, wsgu_ref[...], preferred_element_type=f32)
    gs = gu[:, :_D_FF]
    us = gu[:, _D_FF:]
    hs = (gs * jax.nn.sigmoid(gs)) * us
    shared = jnp.dot(hs, wsdt_ref[...], preferred_element_type=f32)

    out_ref[...] = routed + shared


def _selectors():
    d, f = _D_MODEL, _D_FF
    r = np.zeros((d, d * f), np.float32)
    for i in range(d):
        r[i, i * f:(i + 1) * f] = 1.0
    s = np.zeros((d * f, f), np.float32)
    for i in range(d):
        s[i * f:(i + 1) * f, :] = np.eye(f, dtype=np.float32)
    q = np.zeros((f, d * f), np.float32)
    for i in range(d):
        q[:, i * f:(i + 1) * f] = np.eye(f, dtype=np.float32)
    sd = np.zeros((d * f, d), np.float32)
    for i in range(d):
        sd[i * f:(i + 1) * f, i] = 1.0
    return jnp.asarray(r), jnp.asarray(s), jnp.asarray(q), jnp.asarray(sd)


def kernel(hidden_states, gate_weight, Wg, Wu, Wd, Wsg, Wsu, Wsd):
    orig_shape = hidden_states.shape
    x2 = hidden_states.reshape(-1, _D_MODEL)
    t = x2.shape[0]

    wg_p = Wg.transpose(0, 2, 1).reshape(_NUM_EXPERTS, _D_MODEL * _D_FF)
    wu_p = Wu.transpose(0, 2, 1).reshape(_NUM_EXPERTS, _D_MODEL * _D_FF)
    wd_p = Wd.reshape(_NUM_EXPERTS, _D_MODEL * _D_FF)
    w_all = jnp.concatenate([wg_p, wu_p, wd_p], axis=1)
    wsgu = jnp.concatenate([Wsg.T, Wsu.T], axis=1)
    r, s, q, sd = _selectors()

    n = _D_MODEL * _D_FF
    grid = (t // _TILE,)
    zero = lambda i: (0, 0)
    out = pl.pallas_call(
        _moe_tile_kernel,
        grid=grid,
        in_specs=[
            pl.BlockSpec((_TILE, _D_MODEL), lambda i: (i, 0)),
            pl.BlockSpec((_D_MODEL, _NUM_EXPERTS), zero),
            pl.BlockSpec((_NUM_EXPERTS, 3 * n), zero),
            pl.BlockSpec((_D_MODEL, n), zero),
            pl.BlockSpec((n, _D_FF), zero),
            pl.BlockSpec((_D_FF, n), zero),
            pl.BlockSpec((n, _D_MODEL), zero),
            pl.BlockSpec((_D_MODEL, 2 * _D_FF), zero),
            pl.BlockSpec((_D_FF, _D_MODEL), zero),
        ],
        out_specs=pl.BlockSpec((_TILE, _D_MODEL), lambda i: (i, 0)),
        out_shape=jax.ShapeDtypeStruct((t, _D_MODEL), jnp.float32),
    )(x2, gate_weight.T, w_all, r, s, q, sd, wsgu, Wsd.T)
    return out.reshape(orig_shape)


# speedup vs baseline: 32.4414x; 1.0118x over previous
import jax
import jax.numpy as jnp
import numpy as np
from jax.experimental import pallas as pl

_NUM_EXPERTS = 64
_D_MODEL = 8
_D_FF = 16
_TILE = 8192


def _moe_tile_kernel(x_ref, gwt_ref, wall_ref, r_ref, s_ref, q_ref, sd_ref,
                     wsgu_ref, wsdt_ref, out_ref):
    f32 = jnp.float32
    x = x_ref[...]

    logits = jnp.dot(x, gwt_ref[...], preferred_element_type=f32)
    m = jnp.max(logits, axis=-1, keepdims=True)
    w = 1.0 / jnp.sum(jnp.exp(logits - m), axis=-1, keepdims=True)
    lane = jax.lax.broadcasted_iota(jnp.int32, logits.shape, 1)
    am = jnp.min(jnp.where(logits == m, lane, _NUM_EXPERTS), axis=-1, keepdims=True)
    onehot = (lane == am).astype(f32)

    wt = jnp.dot(onehot, wall_ref[...], preferred_element_type=f32)
    xx = jnp.dot(x, r_ref[...], preferred_element_type=f32)
    g = jnp.dot(xx * wt[:, 0:128], s_ref[...], preferred_element_type=f32)
    u = jnp.dot(xx * wt[:, 128:256], s_ref[...], preferred_element_type=f32)
    h = (g * jax.nn.sigmoid(g)) * u * w
    hh = jnp.dot(h, q_ref[...], preferred_element_type=f32)
    routed = jnp.dot(hh * wt[:, 256:384], sd_ref[...], preferred_element_type=f32)

    gu = jnp.dot(x, wsgu_ref[...], preferred_element_type=f32)
    gs = gu[:, :_D_FF]
    us = gu[:, _D_FF:]
    hs = (gs * jax.nn.sigmoid(gs)) * us
    shared = jnp.dot(hs, wsdt_ref[...], preferred_element_type=f32)

    out_ref[...] = routed + shared


def _selectors():
    d, f = _D_MODEL, _D_FF
    r = np.zeros((d, d * f), np.float32)
    for i in range(d):
        r[i, i * f:(i + 1) * f] = 1.0
    s = np.zeros((d * f, f), np.float32)
    for i in range(d):
        s[i * f:(i + 1) * f, :] = np.eye(f, dtype=np.float32)
    q = np.zeros((f, d * f), np.float32)
    for i in range(d):
        q[:, i * f:(i + 1) * f] = np.eye(f, dtype=np.float32)
    sd = np.zeros((d * f, d), np.float32)
    for i in range(d):
        sd[i * f:(i + 1) * f, i] = 1.0
    return jnp.asarray(r), jnp.asarray(s), jnp.asarray(q), jnp.asarray(sd)


def kernel(hidden_states, gate_weight, Wg, Wu, Wd, Wsg, Wsu, Wsd):
    orig_shape = hidden_states.shape
    x2 = hidden_states.reshape(-1, _D_MODEL)
    t = x2.shape[0]

    wg_p = Wg.transpose(0, 2, 1).reshape(_NUM_EXPERTS, _D_MODEL * _D_FF)
    wu_p = Wu.transpose(0, 2, 1).reshape(_NUM_EXPERTS, _D_MODEL * _D_FF)
    wd_p = Wd.reshape(_NUM_EXPERTS, _D_MODEL * _D_FF)
    w_all = jnp.concatenate([wg_p, wu_p, wd_p], axis=1)
    wsgu = jnp.concatenate([Wsg.T, Wsu.T], axis=1)
    r, s, q, sd = _selectors()

    n = _D_MODEL * _D_FF
    grid = (t // _TILE,)
    zero = lambda i: (0, 0)
    out = pl.pallas_call(
        _moe_tile_kernel,
        grid=grid,
        in_specs=[
            pl.BlockSpec((_TILE, _D_MODEL), lambda i: (i, 0)),
            pl.BlockSpec((_D_MODEL, _NUM_EXPERTS), zero),
            pl.BlockSpec((_NUM_EXPERTS, 3 * n), zero),
            pl.BlockSpec((_D_MODEL, n), zero),
            pl.BlockSpec((n, _D_FF), zero),
            pl.BlockSpec((_D_FF, n), zero),
            pl.BlockSpec((n, _D_MODEL), zero),
            pl.BlockSpec((_D_MODEL, 2 * _D_FF), zero),
            pl.BlockSpec((_D_FF, _D_MODEL), zero),
        ],
        out_specs=pl.BlockSpec((_TILE, _D_MODEL), lambda i: (i, 0)),
        out_shape=jax.ShapeDtypeStruct((t, _D_MODEL), jnp.float32),
    )(x2, gate_weight.T, w_all, r, s, q, sd, wsgu, Wsd.T)
    return out.reshape(orig_shape)
